# Initial kernel scaffold; baseline (speedup 1.0000x reference)
import jax
import jax.numpy as jnp
from jax.experimental import pallas as pl


def kernel(x, edge_index, filter_coeff, weight, bias):
    raise NotImplementedError("write your pallas kernel here")


# speedup vs baseline: 12.0431x; 12.0431x over previous
import functools

import jax
import jax.numpy as jnp
from jax import lax
from jax.experimental import pallas as pl
from jax.experimental.pallas import tpu as pltpu
from jax.experimental.pallas import tpu_sc as plsc

_BATCH = 128
_NTILES = 32


def _zero_fill_2d(ref, nrows, ncols):
    zv = jnp.zeros((16,), jnp.float32)

    def body(i, carry):
        for j in range(ncols // 16):
            ref[i, pl.ds(j * 16, 16)] = zv
        return carry

    lax.fori_loop(0, nrows, body, 0)


def _make_deg_kernel(E, NPAD):
    nb = E // _BATCH
    iters = (nb + _NTILES - 1) // _NTILES
    per_sub = NPAD // 16
    mesh = plsc.VectorSubcoreMesh(core_axis_name="c", subcore_axis_name="s")

    @functools.partial(
        pl.kernel,
        out_type=jax.ShapeDtypeStruct((2, NPAD), jnp.float32),
        mesh=mesh,
        scratch_types=[
            pltpu.VMEM((_BATCH,), jnp.int32),
            pltpu.VMEM((_BATCH,), jnp.float32),
            pltpu.VMEM((per_sub,), jnp.float32),
            pltpu.VMEM_SHARED((NPAD,), jnp.float32),
        ],
    )
    def deg_kernel(dst_hbm, out_hbm, idx_d, ones_v, zbuf, acc):
        c = lax.axis_index("c")
        s = lax.axis_index("s")
        wid = s * 2 + c

        one = jnp.full((16,), 1.0, jnp.float32)
        zero = jnp.zeros((16,), jnp.float32)
        for j in range(_BATCH // 16):
            ones_v[pl.ds(j * 16, 16)] = one

        def zb(i, carry):
            zbuf[pl.ds(i * 16, 16)] = zero
            return carry

        lax.fori_loop(0, per_sub // 16, zb, 0)
        pltpu.sync_copy(zbuf, acc.at[pl.ds(s * per_sub, per_sub)])
        plsc.subcore_barrier()

        def body(i, carry):
            b = wid + _NTILES * i

            @pl.when(b < nb)
            def _():
                pltpu.sync_copy(dst_hbm.at[pl.ds(b * _BATCH, _BATCH)], idx_d)
                pltpu.sync_copy(ones_v, acc.at[idx_d], add=True)

            return carry

        lax.fori_loop(0, iters, body, 0)
        plsc.subcore_barrier()
        pltpu.sync_copy(acc.at[pl.ds(s * per_sub, per_sub)],
                        out_hbm.at[c, pl.ds(s * per_sub, per_sub)])

    return deg_kernel


def _make_spmm_kernel(N, D, E, NR):
    nb = E // _BATCH
    iters = (nb + _NTILES - 1) // _NTILES
    per_sub = NR // 16
    zr = 128
    ncopy = per_sub // zr
    mesh = plsc.VectorSubcoreMesh(core_axis_name="c", subcore_axis_name="s")

    @functools.partial(
        pl.kernel,
        out_type=jax.ShapeDtypeStruct((2, NR, D), jnp.float32),
        mesh=mesh,
        scratch_types=[
            pltpu.VMEM((_BATCH,), jnp.int32),
            pltpu.VMEM((_BATCH,), jnp.int32),
            pltpu.VMEM((_BATCH, D), jnp.float32),
            pltpu.VMEM((zr, D), jnp.float32),
            pltpu.VMEM_SHARED((NR, D), jnp.float32),
            pltpu.SemaphoreType.DMA,
        ],
    )
    def spmm_kernel(x_hbm, src_hbm, dst_hbm, out_hbm,
                    idx_s, idx_d, rows, zbuf, acc, sem):
        c = lax.axis_index("c")
        s = lax.axis_index("s")
        wid = s * 2 + c

        _zero_fill_2d(zbuf, zr, D)
        for j in range(ncopy):
            pltpu.sync_copy(zbuf, acc.at[pl.ds(s * per_sub + j * zr, zr)])
        plsc.subcore_barrier()

        def body(i, carry):
            b = wid + _NTILES * i

            @pl.when(b < nb)
            def _():
                off = b * _BATCH
                pltpu.sync_copy(src_hbm.at[pl.ds(off, _BATCH)], idx_s)
                pltpu.sync_copy(dst_hbm.at[pl.ds(off, _BATCH)], idx_d)
                pltpu.async_copy(x_hbm.at[idx_s], rows, sem).wait()
                pltpu.sync_copy(rows, acc.at[idx_d], add=True)

            return carry

        lax.fori_loop(0, iters, body, 0)
        plsc.subcore_barrier()
        for j in range(ncopy):
            r0 = s * per_sub + j * zr
            pltpu.sync_copy(acc.at[pl.ds(r0, zr)], out_hbm.at[c, pl.ds(r0, zr)])

    return spmm_kernel


def _make_combine(N, D, R):
    def body(x_r, s1_r, p2a_r, p2b_r, cf_r, w_r, b_r, o_r):
        xv = x_r[...]
        s1 = s1_r[...]
        s2 = p2a_r[...] + p2b_r[...]
        cf = cf_r[...]
        acc = jnp.dot(cf[:, 0:1] * xv, w_r[0], preferred_element_type=jnp.float32)
        acc += jnp.dot(cf[:, 1:2] * s1, w_r[1], preferred_element_type=jnp.float32)
        acc += jnp.dot(cf[:, 2:3] * s2, w_r[2], preferred_element_type=jnp.float32)
        acc += jnp.dot(cf[:, 3:4] * xv, w_r[2], preferred_element_type=jnp.float32)
        o_r[...] = acc + b_r[...]

    return pl.pallas_call(
        body,
        out_shape=jax.ShapeDtypeStruct((N, D), jnp.float32),
        grid=(N // R,),
        in_specs=[
            pl.BlockSpec((R, D), lambda i: (i, 0)),
            pl.BlockSpec((R, D), lambda i: (i, 0)),
            pl.BlockSpec((R, D), lambda i: (i, 0)),
            pl.BlockSpec((R, D), lambda i: (i, 0)),
            pl.BlockSpec((R, 4), lambda i: (i, 0)),
            pl.BlockSpec((3, D, D), lambda i: (0, 0, 0)),
            pl.BlockSpec((1, D), lambda i: (0, 0)),
        ],
        out_specs=pl.BlockSpec((R, D), lambda i: (i, 0)),
    )


def kernel(x, edge_index, filter_coeff, weight, bias):
    N, D = x.shape
    E = edge_index.shape[1]
    NPAD = 10240
    src = edge_index[0]
    dst = edge_index[1]

    degp = _make_deg_kernel(E, NPAD)(dst)
    deg = jnp.maximum(degp[0, :N] + degp[1, :N], 1.0)
    dinv = lax.rsqrt(deg)

    spmm = _make_spmm_kernel(N, D, E, NPAD)
    p1 = spmm(dinv[:, None] * x, src, dst)
    s1 = p1[0, :N] + p1[1, :N]
    u2 = (-(dinv * dinv))[:, None] * s1
    p2 = spmm(u2, src, dst)

    fc0, fc1, fc2 = filter_coeff[0], filter_coeff[1], filter_coeff[2]
    coefs = jnp.stack(
        [fc0, -fc1 * dinv, -2.0 * fc2 * dinv, -fc2], axis=1)
    out = _make_combine(N, D, 1000)(
        x, s1, p2[0, :N], p2[1, :N], coefs, weight, bias.reshape(1, D))
    return out


# speedup vs baseline: 24.4716x; 2.0320x over previous
import functools

import jax
import jax.numpy as jnp
from jax import lax
from jax.experimental import pallas as pl
from jax.experimental.pallas import tpu as pltpu
from jax.experimental.pallas import tpu_sc as plsc

_BATCH = 128
_NTILES = 32


def _zero_fill_2d(ref, nrows, ncols):
    zv = jnp.zeros((16,), jnp.float32)

    def body(i, carry):
        for j in range(ncols // 16):
            ref[i, pl.ds(j * 16, 16)] = zv
        return carry

    lax.fori_loop(0, nrows, body, 0)


def _tile_iters(E):
    nb = E // _BATCH
    iters = -(-nb // _NTILES)
    iters += (-iters) % 8
    return nb, iters


def _make_deg_kernel(E, NPAD):
    nb, iters = _tile_iters(E)
    per_sub = NPAD // 16
    mesh = plsc.VectorSubcoreMesh(core_axis_name="c", subcore_axis_name="s")

    @functools.partial(
        pl.kernel,
        out_type=jax.ShapeDtypeStruct((2, NPAD), jnp.float32),
        mesh=mesh,
        scratch_types=[
            pltpu.VMEM((iters, _BATCH), jnp.int32),
            pltpu.VMEM((_BATCH,), jnp.float32),
            pltpu.VMEM((per_sub,), jnp.float32),
            pltpu.VMEM_SHARED((NPAD,), jnp.float32),
            pltpu.SemaphoreType.DMA,
        ],
    )
    def deg_kernel(dst_hbm, out_hbm, idx_d, ones_v, zbuf, acc, sem):
        c = lax.axis_index("c")
        s = lax.axis_index("s")
        wid = s * 2 + c
        start = wid * iters

        pltpu.sync_copy(dst_hbm.at[pl.ds(start, iters)], idx_d)

        one = jnp.full((16,), 1.0, jnp.float32)
        zero = jnp.zeros((16,), jnp.float32)
        for j in range(_BATCH // 16):
            ones_v[pl.ds(j * 16, 16)] = one

        def zb(i, carry):
            zbuf[pl.ds(i * 16, 16)] = zero
            return carry

        lax.fori_loop(0, per_sub // 16, zb, 0)
        pltpu.sync_copy(zbuf, acc.at[pl.ds(s * per_sub, per_sub)])
        plsc.subcore_barrier()

        def body(i, carry):
            @pl.when(start + i < nb)
            def _():
                pltpu.async_copy(ones_v, acc.at[idx_d.at[i]], sem, add=True)

            return carry

        lax.fori_loop(0, iters, body, 0)

        def bodyw(i, carry):
            @pl.when(start + i < nb)
            def _():
                pltpu.make_async_copy(ones_v, acc.at[idx_d.at[i]], sem).wait()

            return carry

        lax.fori_loop(0, iters, bodyw, 0)
        plsc.subcore_barrier()
        pltpu.sync_copy(acc.at[pl.ds(s * per_sub, per_sub)],
                        out_hbm.at[c, pl.ds(s * per_sub, per_sub)])

    return deg_kernel


_CH = 8


def _make_spmm_kernel(N, D, E, NR):
    nb, iters = _tile_iters(E)
    nchunk = iters // _CH
    per_sub = NR // 16
    zr = _BATCH
    ncopy = per_sub // zr
    mesh = plsc.VectorSubcoreMesh(core_axis_name="c", subcore_axis_name="s")

    @functools.partial(
        pl.kernel,
        out_type=jax.ShapeDtypeStruct((2, NR, D), jnp.float32),
        mesh=mesh,
        scratch_types=[
            pltpu.VMEM((2, _CH, _BATCH), jnp.int32),
            pltpu.VMEM((2, _CH, _BATCH), jnp.int32),
            pltpu.VMEM((2, _BATCH, D), jnp.float32),
            pltpu.VMEM_SHARED((NR, D), jnp.float32),
            pltpu.SemaphoreType.DMA,
            pltpu.SemaphoreType.DMA,
        ],
    )
    def spmm_kernel(x_hbm, src_hbm, dst_hbm, out_hbm,
                    idx_s, idx_d, rows, acc, sem0, sem1):
        sems = (sem0, sem1)
        c = lax.axis_index("c")
        s = lax.axis_index("s")
        wid = s * 2 + c
        start = wid * iters

        _zero_fill_2d(rows.at[0], zr, D)
        for j in range(ncopy):
            pltpu.sync_copy(rows.at[0], acc.at[pl.ds(s * per_sub + j * zr, zr)])
        plsc.subcore_barrier()

        def refill(ck, islot):
            def do():
                r0 = start + ck * _CH
                pltpu.sync_copy(src_hbm.at[pl.ds(r0, _CH)], idx_s.at[islot])
                pltpu.sync_copy(dst_hbm.at[pl.ds(r0, _CH)], idx_d.at[islot])

            if isinstance(ck, int):
                if ck < nchunk:
                    do()
            else:
                pl.when(ck < nchunk)(do)

        def fire(i, u, islot, slot):
            @pl.when(jnp.logical_and(i < iters, start + i < nb))
            def _():
                pltpu.async_copy(x_hbm.at[idx_s.at[islot, u]], rows.at[slot],
                                 sems[slot])

        def drain(i, u, islot, slot):
            @pl.when(jnp.logical_and(i < iters, start + i < nb))
            def _():
                pltpu.make_async_copy(x_hbm.at[idx_s.at[islot, u]],
                                      rows.at[slot], sems[slot]).wait()
                pltpu.sync_copy(rows.at[slot], acc.at[idx_d.at[islot, u]],
                                add=True)

        def inner(cbase, k2, islot):
            c0 = cbase + 2 * k2
            i0 = c0 * _CH
            for u in range(_CH):
                i = i0 + u
                if u + 1 < _CH:
                    fire(i + 1, u + 1, islot, (u + 1) % 2)
                else:
                    fire(i + 1, 0, 1 - islot, 0)
                drain(i, u, islot, u % 2)

        refill(0, 0)
        refill(1, 1)
        fire(0, 0, 0, 0)

        def body(k2, carry):
            inner(0, k2, 0)
            refill(2 * k2 + 2, 0)
            inner(1, k2, 1)
            refill(2 * k2 + 3, 1)
            return carry

        lax.fori_loop(0, nchunk // 2, body, 0)

        plsc.subcore_barrier()
        for j in range(ncopy):
            r0 = s * per_sub + j * zr
            pltpu.sync_copy(acc.at[pl.ds(r0, zr)], out_hbm.at[c, pl.ds(r0, zr)])

    return spmm_kernel


def _make_combine(N, D, R):
    def body(x_r, s1_r, p2a_r, p2b_r, cf_r, w_r, b_r, o_r):
        xv = x_r[...]
        s1 = s1_r[...]
        s2 = p2a_r[...] + p2b_r[...]
        cf = cf_r[...]
        acc = jnp.dot(cf[:, 0:1] * xv, w_r[0], preferred_element_type=jnp.float32)
        acc += jnp.dot(cf[:, 1:2] * s1, w_r[1], preferred_element_type=jnp.float32)
        acc += jnp.dot(cf[:, 2:3] * s2, w_r[2], preferred_element_type=jnp.float32)
        acc += jnp.dot(cf[:, 3:4] * xv, w_r[2], preferred_element_type=jnp.float32)
        o_r[...] = acc + b_r[...]

    return pl.pallas_call(
        body,
        out_shape=jax.ShapeDtypeStruct((N, D), jnp.float32),
        grid=(N // R,),
        in_specs=[
            pl.BlockSpec((R, D), lambda i: (i, 0)),
            pl.BlockSpec((R, D), lambda i: (i, 0)),
            pl.BlockSpec((R, D), lambda i: (i, 0)),
            pl.BlockSpec((R, D), lambda i: (i, 0)),
            pl.BlockSpec((R, 4), lambda i: (i, 0)),
            pl.BlockSpec((3, D, D), lambda i: (0, 0, 0)),
            pl.BlockSpec((1, D), lambda i: (0, 0)),
        ],
        out_specs=pl.BlockSpec((R, D), lambda i: (i, 0)),
    )


def kernel(x, edge_index, filter_coeff, weight, bias):
    N, D = x.shape
    E = edge_index.shape[1]
    NPAD = 10240
    nb, iters = _tile_iters(E)
    pad = iters * _NTILES * _BATCH - E
    ei = jnp.pad(edge_index, ((0, 0), (0, pad)))
    src = ei[0].reshape(-1, _BATCH)
    dst = ei[1].reshape(-1, _BATCH)

    degp = _make_deg_kernel(E, NPAD)(dst)
    deg = jnp.maximum(degp[0, :N] + degp[1, :N], 1.0)
    dinv = lax.rsqrt(deg)

    spmm = _make_spmm_kernel(N, D, E, NPAD)
    p1 = spmm(dinv[:, None] * x, src, dst)
    s1 = p1[0, :N] + p1[1, :N]
    u2 = (-(dinv * dinv))[:, None] * s1
    p2 = spmm(u2, src, dst)

    fc0, fc1, fc2 = filter_coeff[0], filter_coeff[1], filter_coeff[2]
    coefs = jnp.stack(
        [fc0, -fc1 * dinv, -2.0 * fc2 * dinv, -fc2], axis=1)
    out = _make_combine(N, D, 1000)(
        x, s1, p2[0, :N], p2[1, :N], coefs, weight, bias.reshape(1, D))
    return out


# speedup vs baseline: 25.1308x; 1.0269x over previous
import functools

import jax
import jax.numpy as jnp
from jax import lax
from jax.experimental import pallas as pl
from jax.experimental.pallas import tpu as pltpu
from jax.experimental.pallas import tpu_sc as plsc

_BATCH = 128
_NTILES = 32


def _zero_fill_2d(ref, nrows, ncols):
    zv = jnp.zeros((16,), jnp.float32)

    def body(i, carry):
        for j in range(ncols // 16):
            ref[i, pl.ds(j * 16, 16)] = zv
        return carry

    lax.fori_loop(0, nrows, body, 0)


def _tile_iters(E):
    nb = E // _BATCH
    iters = -(-nb // _NTILES)
    iters += (-iters) % 8
    return nb, iters


def _make_deg_kernel(E, NPAD):
    nb, iters = _tile_iters(E)
    per_sub = NPAD // 16
    mesh = plsc.VectorSubcoreMesh(core_axis_name="c", subcore_axis_name="s")

    @functools.partial(
        pl.kernel,
        out_type=jax.ShapeDtypeStruct((2, NPAD), jnp.float32),
        mesh=mesh,
        scratch_types=[
            pltpu.VMEM((iters, _BATCH), jnp.int32),
            pltpu.VMEM((_BATCH,), jnp.float32),
            pltpu.VMEM((per_sub,), jnp.float32),
            pltpu.VMEM_SHARED((NPAD,), jnp.float32),
            pltpu.SemaphoreType.DMA,
        ],
    )
    def deg_kernel(dst_hbm, out_hbm, idx_d, ones_v, zbuf, acc, sem):
        c = lax.axis_index("c")
        s = lax.axis_index("s")
        wid = s * 2 + c
        start = wid * iters

        pltpu.sync_copy(dst_hbm.at[pl.ds(start, iters)], idx_d)

        one = jnp.full((16,), 1.0, jnp.float32)
        zero = jnp.zeros((16,), jnp.float32)
        for j in range(_BATCH // 16):
            ones_v[pl.ds(j * 16, 16)] = one

        def zb(i, carry):
            zbuf[pl.ds(i * 16, 16)] = zero
            return carry

        lax.fori_loop(0, per_sub // 16, zb, 0)
        pltpu.sync_copy(zbuf, acc.at[pl.ds(s * per_sub, per_sub)])
        plsc.subcore_barrier()

        def body(i, carry):
            @pl.when(start + i < nb)
            def _():
                pltpu.async_copy(ones_v, acc.at[idx_d.at[i]], sem, add=True)

            return carry

        lax.fori_loop(0, iters, body, 0)

        def bodyw(i, carry):
            @pl.when(start + i < nb)
            def _():
                pltpu.make_async_copy(ones_v, acc.at[idx_d.at[i]], sem).wait()

            return carry

        lax.fori_loop(0, iters, bodyw, 0)
        plsc.subcore_barrier()
        pltpu.sync_copy(acc.at[pl.ds(s * per_sub, per_sub)],
                        out_hbm.at[c, pl.ds(s * per_sub, per_sub)])

    return deg_kernel


_CH = 8


def _make_spmm_kernel(N, D, E, NR):
    nb, iters = _tile_iters(E)
    nchunk = iters // _CH
    per_sub = NR // 16
    zr = _BATCH
    ncopy = per_sub // zr
    mesh = plsc.VectorSubcoreMesh(core_axis_name="c", subcore_axis_name="s")

    @functools.partial(
        pl.kernel,
        out_type=jax.ShapeDtypeStruct((2, NR, D), jnp.float32),
        mesh=mesh,
        scratch_types=[
            pltpu.VMEM((2, _CH, _BATCH), jnp.int32),
            pltpu.VMEM((2, _CH, _BATCH), jnp.int32),
            pltpu.VMEM((2, _BATCH), jnp.int32),
            pltpu.VMEM((2, _BATCH, D), jnp.float32),
            pltpu.VMEM_SHARED((NR, D), jnp.float32),
            pltpu.SemaphoreType.DMA,
            pltpu.SemaphoreType.DMA,
            pltpu.SemaphoreType.DMA,
            pltpu.SemaphoreType.DMA,
        ],
    )
    def spmm_kernel(x_hbm, src_hbm, dst_hbm, out_hbm,
                    idx_s, idx_d, tmp_d, rows, acc, g0, g1, t0, t1):
        gsems = (g0, g1)
        tsems = (t0, t1)
        c = lax.axis_index("c")
        s = lax.axis_index("s")
        wid = s * 2 + c
        start = wid * iters

        _zero_fill_2d(rows.at[0], zr, D)
        for j in range(ncopy):
            pltpu.sync_copy(rows.at[0], acc.at[pl.ds(s * per_sub + j * zr, zr)])
        plsc.subcore_barrier()

        def valid(i):
            return jnp.logical_and(i < iters, start + i < nb)

        def refill(ck, islot):
            def do():
                r0 = start + ck * _CH
                pltpu.sync_copy(src_hbm.at[pl.ds(r0, _CH)], idx_s.at[islot])
                pltpu.sync_copy(dst_hbm.at[pl.ds(r0, _CH)], idx_d.at[islot])

            if isinstance(ck, int):
                if ck < nchunk:
                    do()
            else:
                pl.when(ck < nchunk)(do)

        def scat_wait(slot):
            pltpu.make_async_copy(rows.at[slot], acc.at[tmp_d.at[slot]],
                                  tsems[slot]).wait()

        def fire(i, u, islot, slot):
            if isinstance(i, int) and i < 2:
                pass
            else:
                pl.when(jnp.logical_and(i >= 2, valid(i - 2)))(
                    lambda: scat_wait(slot))

            @pl.when(valid(i))
            def _():
                pltpu.async_copy(x_hbm.at[idx_s.at[islot, u]], rows.at[slot],
                                 gsems[slot])

        def drain(i, u, islot, slot):
            @pl.when(valid(i))
            def _():
                pltpu.make_async_copy(x_hbm.at[idx_s.at[islot, u]],
                                      rows.at[slot], gsems[slot]).wait()
                for q in range(_BATCH // 16):
                    tmp_d[slot, pl.ds(16 * q, 16)] = \
                        idx_d[islot, u, pl.ds(16 * q, 16)]
                pltpu.async_copy(rows.at[slot], acc.at[tmp_d.at[slot]],
                                 tsems[slot], add=True)

        def inner(cbase, k2, islot):
            c0 = cbase + 2 * k2
            i0 = c0 * _CH
            for u in range(_CH):
                i = i0 + u
                if u + 1 < _CH:
                    fire(i + 1, u + 1, islot, (u + 1) % 2)
                else:
                    fire(i + 1, 0, 1 - islot, 0)
                drain(i, u, islot, u % 2)

        refill(0, 0)
        refill(1, 1)
        fire(0, 0, 0, 0)

        def body(k2, carry):
            inner(0, k2, 0)
            refill(2 * k2 + 2, 0)
            inner(1, k2, 1)
            refill(2 * k2 + 3, 1)
            return carry

        lax.fori_loop(0, nchunk // 2, body, 0)

        pl.when(valid(iters - 1))(lambda: scat_wait((iters - 1) % 2))

        plsc.subcore_barrier()
        for j in range(ncopy):
            r0 = s * per_sub + j * zr
            pltpu.sync_copy(acc.at[pl.ds(r0, zr)], out_hbm.at[c, pl.ds(r0, zr)])

    return spmm_kernel


def _make_combine(N, D, R):
    def body(x_r, s1_r, p2a_r, p2b_r, cf_r, w_r, b_r, o_r):
        xv = x_r[...]
        s1 = s1_r[...]
        s2 = p2a_r[...] + p2b_r[...]
        cf = cf_r[...]
        acc = jnp.dot(cf[:, 0:1] * xv, w_r[0], preferred_element_type=jnp.float32)
        acc += jnp.dot(cf[:, 1:2] * s1, w_r[1], preferred_element_type=jnp.float32)
        acc += jnp.dot(cf[:, 2:3] * s2, w_r[2], preferred_element_type=jnp.float32)
        acc += jnp.dot(cf[:, 3:4] * xv, w_r[2], preferred_element_type=jnp.float32)
        o_r[...] = acc + b_r[...]

    return pl.pallas_call(
        body,
        out_shape=jax.ShapeDtypeStruct((N, D), jnp.float32),
        grid=(N // R,),
        in_specs=[
            pl.BlockSpec((R, D), lambda i: (i, 0)),
            pl.BlockSpec((R, D), lambda i: (i, 0)),
            pl.BlockSpec((R, D), lambda i: (i, 0)),
            pl.BlockSpec((R, D), lambda i: (i, 0)),
            pl.BlockSpec((R, 4), lambda i: (i, 0)),
            pl.BlockSpec((3, D, D), lambda i: (0, 0, 0)),
            pl.BlockSpec((1, D), lambda i: (0, 0)),
        ],
        out_specs=pl.BlockSpec((R, D), lambda i: (i, 0)),
    )


def kernel(x, edge_index, filter_coeff, weight, bias):
    N, D = x.shape
    E = edge_index.shape[1]
    NPAD = 10240
    nb, iters = _tile_iters(E)
    pad = iters * _NTILES * _BATCH - E
    ei = jnp.pad(edge_index, ((0, 0), (0, pad)))
    src = ei[0].reshape(-1, _BATCH)
    dst = ei[1].reshape(-1, _BATCH)

    degp = _make_deg_kernel(E, NPAD)(dst)
    deg = jnp.maximum(degp[0, :N] + degp[1, :N], 1.0)
    dinv = lax.rsqrt(deg)

    spmm = _make_spmm_kernel(N, D, E, NPAD)
    p1 = spmm(dinv[:, None] * x, src, dst)
    s1 = p1[0, :N] + p1[1, :N]
    u2 = (-(dinv * dinv))[:, None] * s1
    p2 = spmm(u2, src, dst)

    fc0, fc1, fc2 = filter_coeff[0], filter_coeff[1], filter_coeff[2]
    coefs = jnp.stack(
        [fc0, -fc1 * dinv, -2.0 * fc2 * dinv, -fc2], axis=1)
    out = _make_combine(N, D, 1000)(
        x, s1, p2[0, :N], p2[1, :N], coefs, weight, bias.reshape(1, D))
    return out
